# baseline (speedup 1.0000x reference)
import functools

import jax
import jax.numpy as jnp
from jax import lax
from jax.experimental import pallas as pl
from jax.experimental.pallas import tpu as pltpu
from jax.experimental.pallas import tpu_sc as plsc

DIM = 64
SUB = 128
NSUB = 4
CHUNK = SUB * NSUB
NC, NS = 2, 16
NW = NC * NS


@functools.partial(jax.jit, static_argnames=("n_rows",))
def _sc_gather(idx2d, emb, n_rows):
    rows_per_w = n_rows // NW
    n_chunks = rows_per_w // NSUB
    mesh = plsc.VectorSubcoreMesh(core_axis_name="c", subcore_axis_name="s")

    @functools.partial(
        pl.kernel,
        out_type=jax.ShapeDtypeStruct((n_rows, SUB, DIM), jnp.float32),
        mesh=mesh,
        compiler_params=pltpu.CompilerParams(use_tc_tiling_on_sc=False),
        scratch_types=[
            pltpu.VMEM((NSUB, SUB), jnp.int32),
            pltpu.VMEM((NSUB, SUB, DIM), jnp.float32),
            pltpu.SemaphoreType.DMA,
        ],
    )
    def k(idx_hbm, emb_hbm, out_hbm, idx_v, rows_v, gsem):
        wid = lax.axis_index("s") * NC + lax.axis_index("c")
        base = wid * rows_per_w

        def body(g, carry):
            r0 = base + g * NSUB
            pltpu.sync_copy(idx_hbm.at[pl.ds(r0, NSUB)], idx_v)
            copies = [
                pltpu.async_copy(emb_hbm.at[idx_v.at[j]], rows_v.at[j], gsem)
                for j in range(NSUB)
            ]
            for c in copies:
                c.wait()
            pltpu.sync_copy(rows_v, out_hbm.at[pl.ds(r0, NSUB)])
            return carry

        lax.fori_loop(0, n_chunks, body, 0)

    return k(idx2d, emb)


def kernel(x, emb):
    b, s = x.shape
    n_rows = (b * s) // SUB
    idx2d = x.astype(jnp.int32).reshape(n_rows, SUB)
    out = _sc_gather(idx2d, emb, n_rows)
    return out.reshape(b, s, DIM)


# speedup vs baseline: 1.0424x; 1.0424x over previous
import functools

import jax
import jax.numpy as jnp
from jax import lax
from jax.experimental import pallas as pl
from jax.experimental.pallas import tpu as pltpu
from jax.experimental.pallas import tpu_sc as plsc

DIM = 64
SUB = 128
NSUB = 4
NC, NS = 2, 16
NW = NC * NS


@functools.partial(jax.jit, static_argnames=("n_rows",))
def _sc_gather(idx2d, emb, n_rows):
    rpw = n_rows // NW
    n_half = rpw // NSUB // 2
    mesh = plsc.VectorSubcoreMesh(core_axis_name="c", subcore_axis_name="s")

    @functools.partial(
        pl.kernel,
        out_type=jax.ShapeDtypeStruct((n_rows, SUB, DIM), jnp.float32),
        mesh=mesh,
        compiler_params=pltpu.CompilerParams(use_tc_tiling_on_sc=False),
        scratch_types=[
            pltpu.VMEM((rpw, SUB), jnp.int32),
            pltpu.VMEM((2, NSUB, SUB, DIM), jnp.float32),
            pltpu.SemaphoreType.DMA,
            pltpu.SemaphoreType.DMA,
            pltpu.SemaphoreType.DMA,
            pltpu.SemaphoreType.DMA,
        ],
    )
    def k(idx_hbm, emb_hbm, out_hbm, idx_all, rows_v, gsem0, gsem1, osem0, osem1):
        wid = lax.axis_index("s") * NC + lax.axis_index("c")
        base = wid * rpw
        pltpu.sync_copy(idx_hbm.at[pl.ds(base, rpw)], idx_all)

        r0 = rows_v.at[0]
        r1 = rows_v.at[1]

        def fire_gathers(gl, buf, sem):
            for j in range(NSUB):
                pltpu.async_copy(emb_hbm.at[idx_all.at[gl * NSUB + j]], buf.at[j], sem)

        def fire_store(gl, buf, sem):
            pltpu.async_copy(buf, out_hbm.at[pl.ds(base + gl * NSUB, NSUB)], sem)

        def wait_bytes(buf, sem):
            pltpu.make_async_copy(buf, out_hbm.at[pl.ds(base, NSUB)], sem).wait()

        fire_gathers(0, r0, gsem0)

        def body2(t, carry):
            g0 = 2 * t

            @pl.when(t > 0)
            def _():
                wait_bytes(r1, osem1)
            fire_gathers(g0 + 1, r1, gsem1)
            wait_bytes(r0, gsem0)
            fire_store(g0, r0, osem0)

            @pl.when(t + 1 < n_half)
            def _():
                wait_bytes(r0, osem0)
                fire_gathers(g0 + 2, r0, gsem0)
            wait_bytes(r1, gsem1)
            fire_store(g0 + 1, r1, osem1)
            return carry

        lax.fori_loop(0, n_half, body2, 0)
        wait_bytes(r0, osem0)
        wait_bytes(r1, osem1)

    return k(idx2d, emb)


def kernel(x, emb):
    b, s = x.shape
    n_rows = (b * s) // SUB
    idx2d = x.astype(jnp.int32).reshape(n_rows, SUB)
    out = _sc_gather(idx2d, emb, n_rows)
    return out.reshape(b, s, DIM)
